# Initial kernel scaffold; baseline (speedup 1.0000x reference)
import jax
import jax.numpy as jnp
from jax.experimental import pallas as pl


def kernel(costh, label):
    raise NotImplementedError("write your pallas kernel here")


# speedup vs baseline: 1.8328x; 1.8328x over previous
import jax
import jax.numpy as jnp
from jax.experimental import pallas as pl

_MARGIN = 0.3
_S = 15.0
_B = 1024
_C = 100000
_RB = 8


def _body(costh_ref, lab_ref, out_ref):
    i = pl.program_id(0)
    x = costh_ref[...]
    lab = lab_ref[...]
    cols = jax.lax.broadcasted_iota(jnp.int32, (_RB, _C), 1)
    is_lab = cols == lab
    logits = x * _S
    logits = jnp.where(is_lab, logits - _S * _MARGIN, logits)
    m = jnp.max(logits, axis=1, keepdims=True)
    sexp = jnp.sum(jnp.exp(logits - m), axis=1)
    logz = m[:, 0] + jnp.log(sexp)
    true_logit = jnp.sum(jnp.where(is_lab, logits, 0.0), axis=1)
    part = jnp.sum(logz - true_logit).reshape(1, 1)

    @pl.when(i == 0)
    def _init():
        out_ref[...] = jnp.zeros((1, 1), jnp.float32)

    out_ref[...] += part


def kernel(costh, label):
    lab2d = label.reshape(_B, 1).astype(jnp.int32)
    total = pl.pallas_call(
        _body,
        grid=(_B // _RB,),
        in_specs=[
            pl.BlockSpec((_RB, _C), lambda i: (i, 0)),
            pl.BlockSpec((_RB, 1), lambda i: (i, 0)),
        ],
        out_specs=pl.BlockSpec((1, 1), lambda i: (0, 0)),
        out_shape=jax.ShapeDtypeStruct((1, 1), jnp.float32),
    )(costh, lab2d)
    return total[0, 0] / _B
